# baseline (speedup 1.0000x reference)
import functools

import jax
import jax.numpy as jnp
from jax import lax
from jax.experimental import pallas as pl
from jax.experimental.pallas import tpu as pltpu
from jax.experimental.pallas import tpu_sc as plsc

B = 16384
DIM = 64
NC = 2
NS = 16
NW = NC * NS
SPW = B // NW
CHUNK = 64
NCHUNK = SPW // CHUNK
GROUPS = CHUNK // 16
EPS2 = 1e-24


def _nrsqrt(x):
    i = plsc.bitcast(x, jnp.int32)
    i = jnp.int32(0x5F3759DF) - jnp.right_shift(i, 1)
    y = plsc.bitcast(i, jnp.float32)
    for _ in range(3):
        y = y * (1.5 - 0.5 * x * y * y)
    return y


def _sc_body(h_hbm, r_hbm, t_hbm, ent_hbm, rel_hbm, nrm_hbm,
             res_hbm, part_hbm,
             hidx_v, tidx_v, ridx_v, rdiv_v,
             hcol_v0, tcol_v0, rrows_v0, wrows_v0,
             hcol_v1, tcol_v1, rrows_v1, wrows_v1,
             res_v, pbuf_v, sem_rows0, sem_g0, sem_rows1, sem_g1):
    wid = lax.axis_index("s") * NC + lax.axis_index("c")
    base = wid * SPW
    iota = lax.iota(jnp.int32, 16)
    bufs = [(hcol_v0, tcol_v0, rrows_v0, wrows_v0, sem_rows0, sem_g0),
            (hcol_v1, tcol_v1, rrows_v1, wrows_v1, sem_rows1, sem_g1)]

    pltpu.sync_copy(h_hbm.at[pl.ds(base, SPW)], hidx_v)
    pltpu.sync_copy(t_hbm.at[pl.ds(base, SPW)], tidx_v)
    pltpu.sync_copy(r_hbm.at[pl.ds(base, SPW)], ridx_v)

    def div_body(i, _):
        sl = pl.ds(i * 16, 16)
        rdiv_v[sl] = jnp.right_shift(ridx_v[sl], 1)
        return 0

    lax.fori_loop(0, SPW // 16, div_body, 0)

    acc_orth = jnp.zeros((16,), jnp.float32)
    acc_hp2 = jnp.zeros((16,), jnp.float32)
    acc_tp2 = jnp.zeros((16,), jnp.float32)

    def issue(c, hcol_v, tcol_v, rrows_v, wrows_v, sem_rows, sem_g):
        def fetch16_body(k16, _):
            hseg = hidx_v[pl.ds(c * CHUNK + k16 * 16, 16)]
            tseg = tidx_v[pl.ds(c * CHUNK + k16 * 16, 16)]

            def fetch_body(k, _):
                eh = jnp.sum(jnp.where(iota == k, hseg, 0))
                et = jnp.sum(jnp.where(iota == k, tseg, 0))
                kd = k16 * 16 + k
                pltpu.async_copy(
                    ent_hbm.at[jnp.right_shift(eh, 3),
                               pl.ds(lax.bitwise_and(eh, jnp.int32(7)), 1), :],
                    hcol_v.at[pl.ds(kd, 1), :], sem_rows)
                pltpu.async_copy(
                    ent_hbm.at[jnp.right_shift(et, 3),
                               pl.ds(lax.bitwise_and(et, jnp.int32(7)), 1), :],
                    tcol_v.at[pl.ds(kd, 1), :], sem_rows)
                return 0

            lax.fori_loop(0, 16, fetch_body, 0, unroll=4)
            return 0

        lax.fori_loop(0, GROUPS, fetch16_body, 0)

        sl_c = pl.ds(c * CHUNK, CHUNK)
        pltpu.async_copy(rel_hbm.at[rdiv_v.at[sl_c]], rrows_v, sem_g)
        pltpu.async_copy(nrm_hbm.at[rdiv_v.at[sl_c]], wrows_v, sem_g)

    def drain(c, hcol_v, tcol_v, rrows_v, wrows_v, sem_rows, sem_g):
        def drain_body(k, _):
            pltpu.make_async_copy(
                ent_hbm.at[0, pl.ds(0, 1), :],
                hcol_v.at[pl.ds(0, 1), :], sem_rows).wait()
            return 0

        lax.fori_loop(0, 2 * CHUNK, drain_body, 0, unroll=8)
        sl_c = pl.ds(c * CHUNK, CHUNK)
        pltpu.make_async_copy(rel_hbm.at[rdiv_v.at[sl_c]], rrows_v,
                              sem_g).wait()
        pltpu.make_async_copy(nrm_hbm.at[rdiv_v.at[sl_c]], wrows_v,
                              sem_g).wait()

    issue(0, *bufs[0])
    for c in range(NCHUNK):
        if c + 1 < NCHUNK:
            issue(c + 1, *bufs[(c + 1) % 2])
        hcol_v, tcol_v, rrows_v, wrows_v, sem_rows, sem_g = bufs[c % 2]
        drain(c, *bufs[c % 2])

        def group_body(g, caccs):
            o_acc, h_acc, t_acc = caccs
            off16 = c * CHUNK + g * 16
            cb_r = lax.shift_left(
                lax.bitwise_and(ridx_v[pl.ds(off16, 16)], jnp.int32(1)), 6)
            rrow = g * 16 + iota

            def dim_body(j, accs):
                s, p, q, wr, rr, hh, tt, uu, ur = accs
                rot = lax.bitwise_and(iota + j, jnp.int32(DIM - 1))
                hv = plsc.load_gather(hcol_v, [rrow, rot])
                tv = plsc.load_gather(tcol_v, [rrow, rot])
                rv = plsc.load_gather(rrows_v, [rrow, cb_r + rot])
                wv = plsc.load_gather(wrows_v, [rrow, cb_r + rot])
                u = hv - tv
                return (s + wv * wv, p + wv * hv, q + wv * tv,
                        wr + wv * rv, rr + rv * rv, hh + hv * hv,
                        tt + tv * tv, uu + u * u, ur + u * rv)

            z = jnp.zeros((16,), jnp.float32)
            s, p, q, wr, rr, hh, tt, uu, ur = lax.fori_loop(
                0, DIM, dim_body, (z, z, z, z, z, z, z, z, z), unroll=8)

            m2 = jnp.maximum(s, EPS2)
            inv = 1.0 / m2
            minv = _nrsqrt(m2)
            nu = s * inv
            cw = p - q
            d2 = (uu + cw * cw * inv * (nu - 2.0) + rr + 2.0 * ur
                  - 2.0 * cw * wr * inv)
            d2 = jnp.maximum(d2, 0.0)
            res = d2 * _nrsqrt(jnp.maximum(d2, 1e-30))
            res_v[pl.ds(off16, 16)] = res

            o_acc = o_acc + jnp.maximum(wr * minv - 1e-6, 0.0)
            h_acc = h_acc + hh + p * p * inv * (nu - 2.0)
            t_acc = t_acc + tt + q * q * inv * (nu - 2.0)
            return (o_acc, h_acc, t_acc)

        acc_orth, acc_hp2, acc_tp2 = lax.fori_loop(
            0, GROUPS, group_body, (acc_orth, acc_hp2, acc_tp2))

    pltpu.sync_copy(res_v, res_hbm.at[pl.ds(base, SPW)])
    pbuf_v[pl.ds(0, 16)] = acc_orth
    pbuf_v[pl.ds(16, 16)] = acc_hp2
    pbuf_v[pl.ds(32, 16)] = acc_tp2
    for quant in range(3):
        pltpu.sync_copy(pbuf_v.at[pl.ds(quant * 16, 16)],
                        part_hbm.at[pl.ds(quant * NW * 16 + wid * 16, 16)])


@jax.jit
def _sc_call(h, r, t, ent_t, rel, nrm):
    mesh = plsc.VectorSubcoreMesh(core_axis_name="c", subcore_axis_name="s",
                                  num_cores=NC, num_subcores=NS)
    return pl.kernel(
        _sc_body,
        out_type=(jax.ShapeDtypeStruct((B,), jnp.float32),
                  jax.ShapeDtypeStruct((3 * NW * 16,), jnp.float32)),
        mesh=mesh,
        scratch_types=[
            pltpu.VMEM((SPW,), jnp.int32),
            pltpu.VMEM((SPW,), jnp.int32),
            pltpu.VMEM((SPW,), jnp.int32),
            pltpu.VMEM((SPW,), jnp.int32),
            pltpu.VMEM((CHUNK, DIM), jnp.float32),
            pltpu.VMEM((CHUNK, DIM), jnp.float32),
            pltpu.VMEM((CHUNK, 2 * DIM), jnp.float32),
            pltpu.VMEM((CHUNK, 2 * DIM), jnp.float32),
            pltpu.VMEM((CHUNK, DIM), jnp.float32),
            pltpu.VMEM((CHUNK, DIM), jnp.float32),
            pltpu.VMEM((CHUNK, 2 * DIM), jnp.float32),
            pltpu.VMEM((CHUNK, 2 * DIM), jnp.float32),
            pltpu.VMEM((SPW,), jnp.float32),
            pltpu.VMEM((48,), jnp.float32),
            pltpu.SemaphoreType.DMA,
            pltpu.SemaphoreType.DMA,
            pltpu.SemaphoreType.DMA,
            pltpu.SemaphoreType.DMA,
        ],
        compiler_params=pltpu.CompilerParams(needs_layout_passes=False),
    )(h, r, t, ent_t, rel, nrm)


def _tc_body(res_ref, part_ref, out_ref):
    p = part_ref[...]
    orth = jnp.sum(p[0:4, :])
    hp2 = jnp.sum(p[4:8, :])
    tp2 = jnp.sum(p[8:12, :])
    pen = orth + jnp.maximum(hp2 - 1.0, 0.0) + jnp.maximum(tp2 - 1.0, 0.0)
    out_ref[...] = res_ref[...] + pen


@jax.jit
def _tc_call(res2d, part2d):
    return pl.pallas_call(
        _tc_body,
        out_shape=jax.ShapeDtypeStruct((B // 128, 128), jnp.float32),
    )(res2d, part2d)


def kernel(h, r, t, emb_entity, emb_relation, emb_normal_vec):
    rel2 = emb_relation.reshape(-1, 2 * DIM)
    nrm2 = emb_normal_vec.reshape(-1, 2 * DIM)
    ent3 = emb_entity.reshape(-1, 8, DIM)
    res_raw, partials = _sc_call(h, r, t, ent3, rel2, nrm2)
    out2d = _tc_call(res_raw.reshape(B // 128, 128),
                     partials.reshape(12, 128))
    return out2d.reshape(B)


# speedup vs baseline: 1.0194x; 1.0194x over previous
import functools

import jax
import jax.numpy as jnp
from jax import lax
from jax.experimental import pallas as pl
from jax.experimental.pallas import tpu as pltpu
from jax.experimental.pallas import tpu_sc as plsc

B = 16384
DIM = 64
NC = 2
NS = 16
NW = NC * NS
SPW = B // NW
CHUNK = 64
NCHUNK = SPW // CHUNK
GROUPS = CHUNK // 16
EPS2 = 1e-24


def _nrsqrt(x):
    i = plsc.bitcast(x, jnp.int32)
    i = jnp.int32(0x5F3759DF) - jnp.right_shift(i, 1)
    y = plsc.bitcast(i, jnp.float32)
    for _ in range(3):
        y = y * (1.5 - 0.5 * x * y * y)
    return y


def _sc_body(h_hbm, r_hbm, t_hbm, ent_hbm, rel_hbm, nrm_hbm,
             res_hbm, part_hbm,
             hidx_v, tidx_v, ridx_v, rdiv_v,
             hcol_v0, tcol_v0, rrows_v0, wrows_v0,
             hcol_v1, tcol_v1, rrows_v1, wrows_v1,
             res_v, pbuf_v, sem_rows0, sem_g0, sem_rows1, sem_g1):
    wid = lax.axis_index("s") * NC + lax.axis_index("c")
    base = wid * SPW
    iota = lax.iota(jnp.int32, 16)
    bufs = [(hcol_v0, tcol_v0, rrows_v0, wrows_v0, sem_rows0, sem_g0),
            (hcol_v1, tcol_v1, rrows_v1, wrows_v1, sem_rows1, sem_g1)]

    pltpu.sync_copy(h_hbm.at[pl.ds(base, SPW)], hidx_v)
    pltpu.sync_copy(t_hbm.at[pl.ds(base, SPW)], tidx_v)
    pltpu.sync_copy(r_hbm.at[pl.ds(base, SPW)], ridx_v)

    def div_body(i, _):
        sl = pl.ds(i * 16, 16)
        rdiv_v[sl] = jnp.right_shift(ridx_v[sl], 1)
        return 0

    lax.fori_loop(0, SPW // 16, div_body, 0)

    acc_orth = jnp.zeros((16,), jnp.float32)
    acc_hp2 = jnp.zeros((16,), jnp.float32)
    acc_tp2 = jnp.zeros((16,), jnp.float32)

    def issue(c, hcol_v, tcol_v, rrows_v, wrows_v, sem_rows, sem_g):
        def fetch16_body(k16, _):
            hseg = hidx_v[pl.ds(c * CHUNK + k16 * 16, 16)]
            tseg = tidx_v[pl.ds(c * CHUNK + k16 * 16, 16)]

            def fetch_body(k, _):
                eh = jnp.sum(jnp.where(iota == k, hseg, 0))
                et = jnp.sum(jnp.where(iota == k, tseg, 0))
                kd = k16 * 16 + k
                pltpu.async_copy(
                    ent_hbm.at[jnp.right_shift(eh, 3),
                               pl.ds(lax.bitwise_and(eh, jnp.int32(7)), 1), :],
                    hcol_v.at[pl.ds(kd, 1), :], sem_rows)
                pltpu.async_copy(
                    ent_hbm.at[jnp.right_shift(et, 3),
                               pl.ds(lax.bitwise_and(et, jnp.int32(7)), 1), :],
                    tcol_v.at[pl.ds(kd, 1), :], sem_rows)
                return 0

            lax.fori_loop(0, 16, fetch_body, 0)
            return 0

        lax.fori_loop(0, GROUPS, fetch16_body, 0)

        sl_c = pl.ds(c * CHUNK, CHUNK)
        pltpu.async_copy(rel_hbm.at[rdiv_v.at[sl_c]], rrows_v, sem_g)
        pltpu.async_copy(nrm_hbm.at[rdiv_v.at[sl_c]], wrows_v, sem_g)

    def drain(c, hcol_v, tcol_v, rrows_v, wrows_v, sem_rows, sem_g):
        def drain_body(k, _):
            pltpu.make_async_copy(
                ent_hbm.at[0, pl.ds(0, 1), :],
                hcol_v.at[pl.ds(0, 1), :], sem_rows).wait()
            return 0

        lax.fori_loop(0, 2 * CHUNK, drain_body, 0)
        sl_c = pl.ds(c * CHUNK, CHUNK)
        pltpu.make_async_copy(rel_hbm.at[rdiv_v.at[sl_c]], rrows_v,
                              sem_g).wait()
        pltpu.make_async_copy(nrm_hbm.at[rdiv_v.at[sl_c]], wrows_v,
                              sem_g).wait()

    issue(0, *bufs[0])
    for c in range(NCHUNK):
        if c + 1 < NCHUNK:
            issue(c + 1, *bufs[(c + 1) % 2])
        hcol_v, tcol_v, rrows_v, wrows_v, sem_rows, sem_g = bufs[c % 2]
        drain(c, *bufs[c % 2])

        def group_body(g, caccs):
            o_acc, h_acc, t_acc = caccs
            off16 = c * CHUNK + g * 16
            cb_r = lax.shift_left(
                lax.bitwise_and(ridx_v[pl.ds(off16, 16)], jnp.int32(1)), 6)
            rrow = g * 16 + iota

            def dim_body(j, accs):
                s, p, q, wr, rr, hh, tt, uu, ur = accs
                rot = lax.bitwise_and(iota + j, jnp.int32(DIM - 1))
                hv = plsc.load_gather(hcol_v, [rrow, rot])
                tv = plsc.load_gather(tcol_v, [rrow, rot])
                rv = plsc.load_gather(rrows_v, [rrow, cb_r + rot])
                wv = plsc.load_gather(wrows_v, [rrow, cb_r + rot])
                u = hv - tv
                return (s + wv * wv, p + wv * hv, q + wv * tv,
                        wr + wv * rv, rr + rv * rv, hh + hv * hv,
                        tt + tv * tv, uu + u * u, ur + u * rv)

            z = jnp.zeros((16,), jnp.float32)
            s, p, q, wr, rr, hh, tt, uu, ur = lax.fori_loop(
                0, DIM, dim_body, (z, z, z, z, z, z, z, z, z), unroll=2)

            m2 = jnp.maximum(s, EPS2)
            inv = 1.0 / m2
            minv = _nrsqrt(m2)
            nu = s * inv
            cw = p - q
            d2 = (uu + cw * cw * inv * (nu - 2.0) + rr + 2.0 * ur
                  - 2.0 * cw * wr * inv)
            d2 = jnp.maximum(d2, 0.0)
            res = d2 * _nrsqrt(jnp.maximum(d2, 1e-30))
            res_v[pl.ds(off16, 16)] = res

            o_acc = o_acc + jnp.maximum(wr * minv - 1e-6, 0.0)
            h_acc = h_acc + hh + p * p * inv * (nu - 2.0)
            t_acc = t_acc + tt + q * q * inv * (nu - 2.0)
            return (o_acc, h_acc, t_acc)

        acc_orth, acc_hp2, acc_tp2 = lax.fori_loop(
            0, GROUPS, group_body, (acc_orth, acc_hp2, acc_tp2))

    pltpu.sync_copy(res_v, res_hbm.at[pl.ds(base, SPW)])
    pbuf_v[pl.ds(0, 16)] = acc_orth
    pbuf_v[pl.ds(16, 16)] = acc_hp2
    pbuf_v[pl.ds(32, 16)] = acc_tp2
    for quant in range(3):
        pltpu.sync_copy(pbuf_v.at[pl.ds(quant * 16, 16)],
                        part_hbm.at[pl.ds(quant * NW * 16 + wid * 16, 16)])


@jax.jit
def _sc_call(h, r, t, ent_t, rel, nrm):
    mesh = plsc.VectorSubcoreMesh(core_axis_name="c", subcore_axis_name="s",
                                  num_cores=NC, num_subcores=NS)
    return pl.kernel(
        _sc_body,
        out_type=(jax.ShapeDtypeStruct((B,), jnp.float32),
                  jax.ShapeDtypeStruct((3 * NW * 16,), jnp.float32)),
        mesh=mesh,
        scratch_types=[
            pltpu.VMEM((SPW,), jnp.int32),
            pltpu.VMEM((SPW,), jnp.int32),
            pltpu.VMEM((SPW,), jnp.int32),
            pltpu.VMEM((SPW,), jnp.int32),
            pltpu.VMEM((CHUNK, DIM), jnp.float32),
            pltpu.VMEM((CHUNK, DIM), jnp.float32),
            pltpu.VMEM((CHUNK, 2 * DIM), jnp.float32),
            pltpu.VMEM((CHUNK, 2 * DIM), jnp.float32),
            pltpu.VMEM((CHUNK, DIM), jnp.float32),
            pltpu.VMEM((CHUNK, DIM), jnp.float32),
            pltpu.VMEM((CHUNK, 2 * DIM), jnp.float32),
            pltpu.VMEM((CHUNK, 2 * DIM), jnp.float32),
            pltpu.VMEM((SPW,), jnp.float32),
            pltpu.VMEM((48,), jnp.float32),
            pltpu.SemaphoreType.DMA,
            pltpu.SemaphoreType.DMA,
            pltpu.SemaphoreType.DMA,
            pltpu.SemaphoreType.DMA,
        ],
        compiler_params=pltpu.CompilerParams(needs_layout_passes=False),
    )(h, r, t, ent_t, rel, nrm)


def _tc_body(res_ref, part_ref, out_ref):
    p = part_ref[...]
    orth = jnp.sum(p[0:4, :])
    hp2 = jnp.sum(p[4:8, :])
    tp2 = jnp.sum(p[8:12, :])
    pen = orth + jnp.maximum(hp2 - 1.0, 0.0) + jnp.maximum(tp2 - 1.0, 0.0)
    out_ref[...] = res_ref[...] + pen


@jax.jit
def _tc_call(res2d, part2d):
    return pl.pallas_call(
        _tc_body,
        out_shape=jax.ShapeDtypeStruct((B // 128, 128), jnp.float32),
    )(res2d, part2d)


def kernel(h, r, t, emb_entity, emb_relation, emb_normal_vec):
    rel2 = emb_relation.reshape(-1, 2 * DIM)
    nrm2 = emb_normal_vec.reshape(-1, 2 * DIM)
    ent3 = emb_entity.reshape(-1, 8, DIM)
    res_raw, partials = _sc_call(h, r, t, ent3, rel2, nrm2)
    out2d = _tc_call(res_raw.reshape(B // 128, 128),
                     partials.reshape(12, 128))
    return out2d.reshape(B)
